# baseline (speedup 1.0000x reference)
import functools

import jax
import jax.numpy as jnp
from jax import lax
from jax.experimental import pallas as pl
from jax.experimental.pallas import tpu as pltpu
import jax.experimental.pallas.tpu_sc as plsc

N = 10000
NP = 10240
DF = 128
HID = 64
H1 = 8
NCLS = 40
R = 512
GRID = NP // R

T1W = 80
T2W = 48
T2R = 80
ADW = 16

NC = 2
NS = 16
CH = 128
ROWS_PER_TILE = NP // NS
ZR = 64


def _tc1_body(x_ref, w1_ref, am_ref, bm_ref, t1_ref, ad_ref, m_ref):
    i = pl.program_id(0)
    h = jnp.dot(x_ref[...], w1_ref[...], preferred_element_type=jnp.float32)
    asrc = jnp.dot(h, am_ref[...], preferred_element_type=jnp.float32)
    adst = jnp.dot(h, bm_ref[...], preferred_element_type=jnp.float32)
    t1_ref[...] = jnp.concatenate([h, asrc, asrc], axis=1)
    ad_ref[...] = jnp.concatenate([adst, adst], axis=1)
    bm = jnp.concatenate(
        [jnp.max(asrc, axis=0, keepdims=True),
         jnp.max(adst, axis=0, keepdims=True)], axis=1)

    @pl.when(i == 0)
    def _():
        m_ref[...] = bm

    @pl.when(i > 0)
    def _():
        m_ref[...] = jnp.maximum(m_ref[...], bm)


def _tc1(x_pad, w1, am, bm):
    return pl.pallas_call(
        _tc1_body,
        grid=(GRID,),
        in_specs=[
            pl.BlockSpec((R, DF), lambda i: (i, 0)),
            pl.BlockSpec((DF, HID), lambda i: (0, 0)),
            pl.BlockSpec((HID, H1), lambda i: (0, 0)),
            pl.BlockSpec((HID, H1), lambda i: (0, 0)),
        ],
        out_specs=[
            pl.BlockSpec((R, T1W), lambda i: (i, 0)),
            pl.BlockSpec((R, ADW), lambda i: (i, 0)),
            pl.BlockSpec((1, 16), lambda i: (0, 0)),
        ],
        out_shape=[
            jax.ShapeDtypeStruct((NP, T1W), jnp.float32),
            jax.ShapeDtypeStruct((NP, ADW), jnp.float32),
            jax.ShapeDtypeStruct((1, 16), jnp.float32),
        ],
    )(x_pad, w1, am, bm)


def _tc2_body(acc_ref, b1_ref, g1_ref, be1_ref, w2_ref, as2_ref, ad2_ref,
              sel_ref, t2_ref, ad_ref, m_ref):
    i = pl.program_id(0)
    p = acc_ref[0] + acc_ref[1]
    num = p[:, :HID]
    den = p[:, HID:HID + H1]
    rinv = 1.0 / (den + 1e-16)
    rexp = jnp.dot(rinv, sel_ref[...], preferred_element_type=jnp.float32)
    h = num * rexp + b1_ref[...]
    mu = jnp.mean(h, axis=1, keepdims=True)
    var = jnp.mean((h - mu) ** 2, axis=1, keepdims=True)
    hn = (h - mu) * lax.rsqrt(var + 1e-5) * g1_ref[...] + be1_ref[...]
    he = jnp.where(hn > 0, hn, jnp.exp(hn) - 1.0)
    h2 = jnp.dot(he, w2_ref[...], preferred_element_type=jnp.float32)
    s2 = jnp.dot(h2, as2_ref[...], preferred_element_type=jnp.float32)
    d2 = jnp.dot(h2, ad2_ref[...], preferred_element_type=jnp.float32)
    t2_ref[...] = jnp.concatenate(
        [h2, jnp.zeros((R, 8), jnp.float32), s2,
         jnp.zeros((R, 16), jnp.float32)], axis=1)
    ad_ref[...] = d2
    bm = jnp.concatenate(
        [jnp.max(s2[:, :8], axis=0, keepdims=True),
         jnp.max(d2[:, :8], axis=0, keepdims=True)], axis=1)

    @pl.when(i == 0)
    def _():
        m_ref[...] = bm

    @pl.when(i > 0)
    def _():
        m_ref[...] = jnp.maximum(m_ref[...], bm)


def _tc2(pacc, b1, g1, be1, w2, as2t, ad2t, sel):
    return pl.pallas_call(
        _tc2_body,
        grid=(GRID,),
        in_specs=[
            pl.BlockSpec((NC, R, T1W), lambda i: (0, i, 0)),
            pl.BlockSpec((1, HID), lambda i: (0, 0)),
            pl.BlockSpec((1, HID), lambda i: (0, 0)),
            pl.BlockSpec((1, HID), lambda i: (0, 0)),
            pl.BlockSpec((HID, NCLS), lambda i: (0, 0)),
            pl.BlockSpec((NCLS, 16), lambda i: (0, 0)),
            pl.BlockSpec((NCLS, 16), lambda i: (0, 0)),
            pl.BlockSpec((H1, HID), lambda i: (0, 0)),
        ],
        out_specs=[
            pl.BlockSpec((R, T2R), lambda i: (i, 0)),
            pl.BlockSpec((R, ADW), lambda i: (i, 0)),
            pl.BlockSpec((1, 16), lambda i: (0, 0)),
        ],
        out_shape=[
            jax.ShapeDtypeStruct((NP, T2R), jnp.float32),
            jax.ShapeDtypeStruct((NP, ADW), jnp.float32),
            jax.ShapeDtypeStruct((1, 16), jnp.float32),
        ],
    )(pacc, b1, g1, be1, w2, as2t, ad2t, sel)


def _tc3_body(acc_ref, b2_ref, g2_ref, be2_ref, out_ref):
    p = acc_ref[0] + acc_ref[1]
    num = p[:, :NCLS]
    den = p[:, NCLS:NCLS + 1]
    o = num * (1.0 / (den + 1e-16)) + b2_ref[...]
    mu = jnp.mean(o, axis=1, keepdims=True)
    var = jnp.mean((o - mu) ** 2, axis=1, keepdims=True)
    out_ref[...] = (o - mu) * lax.rsqrt(var + 1e-5) * g2_ref[...] + be2_ref[...]


def _tc3(pacc, b2, g2, be2):
    return pl.pallas_call(
        _tc3_body,
        grid=(GRID,),
        in_specs=[
            pl.BlockSpec((NC, R, T2W), lambda i: (0, i, 0)),
            pl.BlockSpec((1, NCLS), lambda i: (0, 0)),
            pl.BlockSpec((1, NCLS), lambda i: (0, 0)),
            pl.BlockSpec((1, NCLS), lambda i: (0, 0)),
        ],
        out_specs=pl.BlockSpec((R, NCLS), lambda i: (i, 0)),
        out_shape=jax.ShapeDtypeStruct((NP, NCLS), jnp.float32),
    )(pacc, b2, g2, be2)


def _zero_acc(acc, zbuf, sid, width):
    def zrow(r, _):
        for c in range(width // 16):
            zbuf[r, pl.ds(16 * c, 16)] = jnp.zeros((16,), jnp.float32)
        return 0

    lax.fori_loop(0, ZR, zrow, 0)
    base = sid * ROWS_PER_TILE
    for k in range(ROWS_PER_TILE // ZR):
        pltpu.sync_copy(zbuf, acc.at[pl.ds(base + k * ZR, ZR)])


def _writeback(acc, out, cid, sid):
    base = sid * ROWS_PER_TILE
    pltpu.sync_copy(acc.at[pl.ds(base, ROWS_PER_TILE)],
                    out.at[cid, pl.ds(base, ROWS_PER_TILE)])


def _sc_mesh():
    return plsc.VectorSubcoreMesh(core_axis_name="c", subcore_axis_name="s")


def _sc_pipeline(nb, rowbase, src2d, dst2d, tab, adt, acc,
                 idxs_all, idxd_all, rows, adst, contrib, semg, semsc,
                 compute_chunk):
    pltpu.sync_copy(src2d.at[pl.ds(rowbase, nb)], idxs_all)
    pltpu.sync_copy(dst2d.at[pl.ds(rowbase, nb)], idxd_all)

    def issue_gather(j, b):
        pltpu.async_copy(tab.at[idxs_all.at[j]], rows.at[b], semg.at[b])
        pltpu.async_copy(adt.at[idxd_all.at[j]], adst.at[b], semg.at[b])

    def wait_gather(j, b):
        pltpu.make_async_copy(tab.at[idxs_all.at[j]], rows.at[b],
                              semg.at[b]).wait()
        pltpu.make_async_copy(adt.at[idxd_all.at[j]], adst.at[b],
                              semg.at[b]).wait()

    def issue_scatter(j, b):
        pltpu.async_copy(contrib.at[b], acc.at[idxd_all.at[j]], semsc.at[b],
                         add=True)

    def wait_scatter(j, b):
        pltpu.make_async_copy(contrib.at[b], acc.at[idxd_all.at[j]],
                              semsc.at[b]).wait()

    issue_gather(0, 0)
    npairs = nb // 2

    def body(k, _):
        a = 2 * k
        issue_gather(a + 1, 1)

        @pl.when(k > 0)
        def _():
            wait_scatter(a - 2, 0)

        wait_gather(a, 0)
        compute_chunk(0)
        issue_scatter(a, 0)

        @pl.when(a + 2 < nb)
        def _():
            issue_gather(a + 2, 0)

        @pl.when(k > 0)
        def _():
            wait_scatter(a - 1, 1)

        wait_gather(a + 1, 1)
        compute_chunk(1)
        issue_scatter(a + 1, 1)
        return 0

    lax.fori_loop(0, npairs, body, 0)
    last = 2 * npairs
    if nb % 2 == 1:
        wait_scatter(last - 2, 0)
        wait_gather(last, 0)
        compute_chunk(0)
        issue_scatter(last, 0)
        wait_scatter(last - 1, 1)
        wait_scatter(last, 0)
    else:
        wait_scatter(last - 2, 0)
        wait_scatter(last - 1, 1)


def _make_sc1(ep):
    per_tile = ep // (NC * NS)
    nb = per_tile // CH

    @functools.partial(
        pl.kernel,
        out_type=jax.ShapeDtypeStruct((NC, NP, T1W), jnp.float32),
        mesh=_sc_mesh(),
        compiler_params=pltpu.CompilerParams(
            needs_layout_passes=False, use_tc_tiling_on_sc=False),
        scratch_types=[
            pltpu.VMEM_SHARED((NP, T1W), jnp.float32),
            pltpu.VMEM((nb, CH), jnp.int32),
            pltpu.VMEM((nb, CH), jnp.int32),
            pltpu.VMEM((2, CH, T1W), jnp.float32),
            pltpu.VMEM((2, CH, ADW), jnp.float32),
            pltpu.VMEM((2, CH, T1W), jnp.float32),
            pltpu.VMEM((16,), jnp.float32),
            pltpu.VMEM((CH, 16), jnp.float32),
            pltpu.VMEM((ZR, T1W), jnp.float32),
            pltpu.SemaphoreType.DMA((2,)),
            pltpu.SemaphoreType.DMA((2,)),
        ],
    )
    def sc1(src_hbm, dst_hbm, t1_hbm, ad_hbm, m_hbm, out_hbm,
            acc, idxs_all, idxd_all, rows, adst, contrib, mv, wtmp, zbuf,
            semg, semsc):
        cid = lax.axis_index("c")
        sid = lax.axis_index("s")
        _zero_acc(acc, zbuf, sid, T1W)

        pltpu.sync_copy(m_hbm, mv)
        io = lax.iota(jnp.int32, 16)
        hio = jnp.where(io < 8, io, io - 8)
        ca = plsc.load_gather(mv, [hio])
        cb = plsc.load_gather(mv, [hio + 8])
        z0 = ca + cb
        cvec = jnp.where(z0 > 0, z0, z0 * 0.2)

        plsc.subcore_barrier()

        def compute_chunk(b):
            @plsc.parallel_loop(0, CH, 1, unroll=4)
            def _(e):
                za = rows[b, e, pl.ds(HID, 16)]
                zb = adst[b, e, pl.ds(0, 16)]
                z = za + zb
                lr = jnp.where(z > 0, z, z * 0.2)
                w = jnp.exp(lr - cvec)
                contrib[b, e, pl.ds(HID, 16)] = w
                for v in range(4):
                    contrib[b, e, pl.ds(16 * v, 16)] = (
                        rows[b, e, pl.ds(16 * v, 16)] * w)

        rowbase = cid * (ep // NC // CH) + sid * nb
        _sc_pipeline(nb, rowbase, src_hbm, dst_hbm, t1_hbm, ad_hbm, acc,
                     idxs_all, idxd_all, rows, adst, contrib, semg, semsc,
                     compute_chunk)
        plsc.subcore_barrier()
        _writeback(acc, out_hbm, cid, sid)

    return sc1


def _make_sc2(ep):
    per_tile = ep // (NC * NS)
    nb = per_tile // CH

    @functools.partial(
        pl.kernel,
        out_type=jax.ShapeDtypeStruct((NC, NP, T2W), jnp.float32),
        mesh=_sc_mesh(),
        compiler_params=pltpu.CompilerParams(
            needs_layout_passes=False, use_tc_tiling_on_sc=False),
        scratch_types=[
            pltpu.VMEM_SHARED((NP, T2W), jnp.float32),
            pltpu.VMEM((nb, CH), jnp.int32),
            pltpu.VMEM((nb, CH), jnp.int32),
            pltpu.VMEM((2, CH, T2R), jnp.float32),
            pltpu.VMEM((2, CH, ADW), jnp.float32),
            pltpu.VMEM((2, CH, T2W), jnp.float32),
            pltpu.VMEM((16,), jnp.float32),
            pltpu.VMEM((CH, 16), jnp.float32),
            pltpu.VMEM((ZR, T2W), jnp.float32),
            pltpu.SemaphoreType.DMA((2,)),
            pltpu.SemaphoreType.DMA((2,)),
        ],
    )
    def sc2(src_hbm, dst_hbm, t2_hbm, ad_hbm, m_hbm, out_hbm,
            acc, idxs_all, idxd_all, rows, adst, contrib, mv, wtmp, zbuf,
            semg, semsc):
        cid = lax.axis_index("c")
        sid = lax.axis_index("s")
        _zero_acc(acc, zbuf, sid, T2W)

        pltpu.sync_copy(m_hbm, mv)
        io = lax.iota(jnp.int32, 16)
        ms = plsc.load_gather(mv, [jnp.zeros((16,), jnp.int32)])
        md = plsc.load_gather(mv, [jnp.full((16,), 8, jnp.int32)])
        z0 = ms + md
        cvec = jnp.where(z0 > 0, z0, z0 * 0.2)

        plsc.subcore_barrier()

        def compute_chunk(b):
            @plsc.parallel_loop(0, CH, 1, unroll=4)
            def _(e):
                za = rows[b, e, pl.ds(48, 16)]
                zb = adst[b, e, pl.ds(0, 16)]
                z = za + zb
                lr = jnp.where(z > 0, z, z * 0.2)
                w0 = jnp.exp(lr - cvec)
                wtmp[e, pl.ds(0, 16)] = w0
                w = plsc.load_gather(wtmp.at[e], [jnp.full((16,), 8, jnp.int32)])
                contrib[b, e, pl.ds(0, 16)] = rows[b, e, pl.ds(0, 16)] * w
                contrib[b, e, pl.ds(16, 16)] = rows[b, e, pl.ds(16, 16)] * w
                v2 = jnp.where(io < 8, rows[b, e, pl.ds(32, 16)] * w, w)
                contrib[b, e, pl.ds(32, 16)] = v2

        rowbase = cid * (ep // NC // CH) + sid * nb
        _sc_pipeline(nb, rowbase, src_hbm, dst_hbm, t2_hbm, ad_hbm, acc,
                     idxs_all, idxd_all, rows, adst, contrib, semg, semsc,
                     compute_chunk)
        plsc.subcore_barrier()
        _writeback(acc, out_hbm, cid, sid)

    return sc2


def kernel(x, edge_index, W1, att_src1, att_dst1, bias1, gamma1, beta1,
           W2, att_src2, att_dst2, bias2, gamma2, beta2):
    n, e = x.shape[0], edge_index.shape[1]
    loop = jnp.arange(n, dtype=edge_index.dtype)
    src = jnp.concatenate([edge_index[0], loop])
    dst = jnp.concatenate([edge_index[1], loop])
    el = e + n
    unit = NC * NS * CH
    ep = ((el + unit - 1) // unit) * unit
    pad = jnp.full((ep - el,), NP - 1, edge_index.dtype)
    srcp = jnp.concatenate([src, pad]).reshape(-1, CH)
    dstp = jnp.concatenate([dst, pad]).reshape(-1, CH)

    x_pad = jnp.pad(x, ((0, NP - n), (0, 0)))

    q = jnp.arange(HID)
    cq = (q % H1) * 8 + q // H1
    selp = ((q % H1)[:, None] ==
            jnp.arange(H1)[None, :]).astype(jnp.float32)
    w1p = W1[:, cq]
    am = att_src1.reshape(HID)[cq][:, None] * selp
    bm = att_dst1.reshape(HID)[cq][:, None] * selp
    selp8 = selp.T
    as2t = jnp.tile(att_src2.reshape(NCLS, 1), (1, 16))
    ad2t = jnp.tile(att_dst2.reshape(NCLS, 1), (1, 16))

    t1, ad1, m1 = _tc1(x_pad, w1p, am, bm)
    pacc1 = _make_sc1(ep)(srcp, dstp, t1, ad1, m1.reshape(16))
    t2, ad2, m2 = _tc2(pacc1, bias1[cq].reshape(1, HID),
                       gamma1[cq].reshape(1, HID),
                       beta1[cq].reshape(1, HID), W2[cq, :], as2t, ad2t,
                       selp8)
    _DBG = False
    if _DBG:
        m2f = m2.reshape(16)
        z02 = m2f[0] + m2f[8]
        c2 = jnp.where(z02 > 0, z02, 0.2 * z02)
        zdbg = t2[:, 48][src] + ad2[:, 0][dst]
        wdbg = jnp.exp(jnp.where(zdbg > 0, zdbg, 0.2 * zdbg) - c2)
        num2 = jax.ops.segment_sum(t2[:, :40][src] * wdbg[:, None], dst,
                                   num_segments=NP)
        den2 = jax.ops.segment_sum(wdbg, dst, num_segments=NP)
        acc2 = jnp.concatenate([num2, jnp.tile(den2[:, None], (1, 8))], axis=1)
        pacc2 = jnp.stack([acc2, jnp.zeros_like(acc2)])
    else:
        pacc2 = _make_sc2(ep)(srcp, dstp, t2, ad2, m2.reshape(16))
    out = _tc3(pacc2, bias2.reshape(1, NCLS), gamma2.reshape(1, NCLS),
               beta2.reshape(1, NCLS))
    return out[:N]


# speedup vs baseline: 1.1190x; 1.1190x over previous
import functools

import jax
import jax.numpy as jnp
from jax import lax
from jax.experimental import pallas as pl
from jax.experimental.pallas import tpu as pltpu
import jax.experimental.pallas.tpu_sc as plsc

N = 10000
NP = 10240
DF = 128
HID = 64
H1 = 8
NCLS = 40
R = 512
GRID = NP // R

T1W = 80
T2W = 48
T2R = 48
ADW = 16

NC = 2
NS = 16
CH = 128
ROWS_PER_TILE = NP // NS
ZR = 64


def _tc1_body(x_ref, w1_ref, am_ref, bm_ref, t1_ref, ad_ref, m_ref):
    i = pl.program_id(0)
    h = jnp.dot(x_ref[...], w1_ref[...], preferred_element_type=jnp.float32)
    asrc = jnp.dot(h, am_ref[...], preferred_element_type=jnp.float32)
    adst = jnp.dot(h, bm_ref[...], preferred_element_type=jnp.float32)
    t1_ref[...] = jnp.concatenate([h, asrc, asrc], axis=1)
    ad_ref[...] = jnp.concatenate([adst, adst], axis=1)
    bm = jnp.concatenate(
        [jnp.max(asrc, axis=0, keepdims=True),
         jnp.max(adst, axis=0, keepdims=True)], axis=1)

    @pl.when(i == 0)
    def _():
        m_ref[...] = bm

    @pl.when(i > 0)
    def _():
        m_ref[...] = jnp.maximum(m_ref[...], bm)


def _tc1(x_pad, w1, am, bm):
    return pl.pallas_call(
        _tc1_body,
        grid=(GRID,),
        in_specs=[
            pl.BlockSpec((R, DF), lambda i: (i, 0)),
            pl.BlockSpec((DF, HID), lambda i: (0, 0)),
            pl.BlockSpec((HID, H1), lambda i: (0, 0)),
            pl.BlockSpec((HID, H1), lambda i: (0, 0)),
        ],
        out_specs=[
            pl.BlockSpec((R, T1W), lambda i: (i, 0)),
            pl.BlockSpec((R, ADW), lambda i: (i, 0)),
            pl.BlockSpec((1, 16), lambda i: (0, 0)),
        ],
        out_shape=[
            jax.ShapeDtypeStruct((NP, T1W), jnp.float32),
            jax.ShapeDtypeStruct((NP, ADW), jnp.float32),
            jax.ShapeDtypeStruct((1, 16), jnp.float32),
        ],
    )(x_pad, w1, am, bm)


def _tc2_body(acc_ref, b1_ref, g1_ref, be1_ref, w2_ref, as2_ref, ad2_ref,
              sel_ref, t2_ref, ad_ref, m_ref):
    i = pl.program_id(0)
    p = acc_ref[0] + acc_ref[1]
    num = p[:, :HID]
    den = p[:, HID:HID + H1]
    rinv = 1.0 / (den + 1e-16)
    rexp = jnp.dot(rinv, sel_ref[...], preferred_element_type=jnp.float32)
    h = num * rexp + b1_ref[...]
    mu = jnp.mean(h, axis=1, keepdims=True)
    var = jnp.mean((h - mu) ** 2, axis=1, keepdims=True)
    hn = (h - mu) * lax.rsqrt(var + 1e-5) * g1_ref[...] + be1_ref[...]
    he = jnp.where(hn > 0, hn, jnp.exp(hn) - 1.0)
    h2 = jnp.dot(he, w2_ref[...], preferred_element_type=jnp.float32)
    s2 = jnp.dot(h2, as2_ref[...], preferred_element_type=jnp.float32)
    d2 = jnp.dot(h2, ad2_ref[...], preferred_element_type=jnp.float32)
    t2_ref[...] = jnp.concatenate([h2, s2[:, :8]], axis=1)
    ad_ref[...] = d2
    bm = jnp.concatenate(
        [jnp.max(s2[:, :8], axis=0, keepdims=True),
         jnp.max(d2[:, :8], axis=0, keepdims=True)], axis=1)

    @pl.when(i == 0)
    def _():
        m_ref[...] = bm

    @pl.when(i > 0)
    def _():
        m_ref[...] = jnp.maximum(m_ref[...], bm)


def _tc2(pacc, b1, g1, be1, w2, as2t, ad2t, sel):
    return pl.pallas_call(
        _tc2_body,
        grid=(GRID,),
        in_specs=[
            pl.BlockSpec((NC, R, T1W), lambda i: (0, i, 0)),
            pl.BlockSpec((1, HID), lambda i: (0, 0)),
            pl.BlockSpec((1, HID), lambda i: (0, 0)),
            pl.BlockSpec((1, HID), lambda i: (0, 0)),
            pl.BlockSpec((HID, NCLS), lambda i: (0, 0)),
            pl.BlockSpec((NCLS, 16), lambda i: (0, 0)),
            pl.BlockSpec((NCLS, 16), lambda i: (0, 0)),
            pl.BlockSpec((H1, HID), lambda i: (0, 0)),
        ],
        out_specs=[
            pl.BlockSpec((R, T2R), lambda i: (i, 0)),
            pl.BlockSpec((R, ADW), lambda i: (i, 0)),
            pl.BlockSpec((1, 16), lambda i: (0, 0)),
        ],
        out_shape=[
            jax.ShapeDtypeStruct((NP, T2R), jnp.float32),
            jax.ShapeDtypeStruct((NP, ADW), jnp.float32),
            jax.ShapeDtypeStruct((1, 16), jnp.float32),
        ],
    )(pacc, b1, g1, be1, w2, as2t, ad2t, sel)


def _tc3_body(acc_ref, b2_ref, g2_ref, be2_ref, out_ref):
    p = acc_ref[0] + acc_ref[1]
    num = p[:, :NCLS]
    den = p[:, NCLS:NCLS + 1]
    o = num * (1.0 / (den + 1e-16)) + b2_ref[...]
    mu = jnp.mean(o, axis=1, keepdims=True)
    var = jnp.mean((o - mu) ** 2, axis=1, keepdims=True)
    out_ref[...] = (o - mu) * lax.rsqrt(var + 1e-5) * g2_ref[...] + be2_ref[...]


def _tc3(pacc, b2, g2, be2):
    return pl.pallas_call(
        _tc3_body,
        grid=(GRID,),
        in_specs=[
            pl.BlockSpec((NC, R, T2W), lambda i: (0, i, 0)),
            pl.BlockSpec((1, NCLS), lambda i: (0, 0)),
            pl.BlockSpec((1, NCLS), lambda i: (0, 0)),
            pl.BlockSpec((1, NCLS), lambda i: (0, 0)),
        ],
        out_specs=pl.BlockSpec((R, NCLS), lambda i: (i, 0)),
        out_shape=jax.ShapeDtypeStruct((NP, NCLS), jnp.float32),
    )(pacc, b2, g2, be2)


def _zero_acc(acc, zbuf, sid, width):
    def zrow(r, _):
        for c in range(width // 16):
            zbuf[r, pl.ds(16 * c, 16)] = jnp.zeros((16,), jnp.float32)
        return 0

    lax.fori_loop(0, ZR, zrow, 0)
    base = sid * ROWS_PER_TILE
    for k in range(ROWS_PER_TILE // ZR):
        pltpu.sync_copy(zbuf, acc.at[pl.ds(base + k * ZR, ZR)])


def _writeback(acc, out, cid, sid):
    base = sid * ROWS_PER_TILE
    pltpu.sync_copy(acc.at[pl.ds(base, ROWS_PER_TILE)],
                    out.at[cid, pl.ds(base, ROWS_PER_TILE)])


def _sc_mesh():
    return plsc.VectorSubcoreMesh(core_axis_name="c", subcore_axis_name="s")


def _sc_pipeline(nb, rowbase, src2d, dst2d, tab, adt, acc,
                 idxs_all, idxd_all, rows, adst, contrib, semg, semsc,
                 compute_chunk):
    pltpu.sync_copy(src2d.at[pl.ds(rowbase, nb)], idxs_all)
    pltpu.sync_copy(dst2d.at[pl.ds(rowbase, nb)], idxd_all)

    def issue_gather(j, b):
        pltpu.async_copy(tab.at[idxs_all.at[j]], rows.at[b], semg.at[b])
        pltpu.async_copy(adt.at[idxd_all.at[j]], adst.at[b], semg.at[b])

    def wait_gather(j, b):
        pltpu.make_async_copy(tab.at[idxs_all.at[j]], rows.at[b],
                              semg.at[b]).wait()
        pltpu.make_async_copy(adt.at[idxd_all.at[j]], adst.at[b],
                              semg.at[b]).wait()

    def issue_scatter(j, b):
        pltpu.async_copy(contrib.at[b], acc.at[idxd_all.at[j]], semsc.at[b],
                         add=True)

    def wait_scatter(j, b):
        pltpu.make_async_copy(contrib.at[b], acc.at[idxd_all.at[j]],
                              semsc.at[b]).wait()

    issue_gather(0, 0)
    npairs = nb // 2

    def body(k, _):
        a = 2 * k
        issue_gather(a + 1, 1)

        @pl.when(k > 0)
        def _():
            wait_scatter(a - 2, 0)

        wait_gather(a, 0)
        compute_chunk(0)
        issue_scatter(a, 0)

        @pl.when(a + 2 < nb)
        def _():
            issue_gather(a + 2, 0)

        @pl.when(k > 0)
        def _():
            wait_scatter(a - 1, 1)

        wait_gather(a + 1, 1)
        compute_chunk(1)
        issue_scatter(a + 1, 1)
        return 0

    lax.fori_loop(0, npairs, body, 0)
    last = 2 * npairs
    if nb % 2 == 1:
        wait_scatter(last - 2, 0)
        wait_gather(last, 0)
        compute_chunk(0)
        issue_scatter(last, 0)
        wait_scatter(last - 1, 1)
        wait_scatter(last, 0)
    else:
        wait_scatter(last - 2, 0)
        wait_scatter(last - 1, 1)


def _make_sc1(ep):
    per_tile = ep // (NC * NS)
    nb = per_tile // CH

    @functools.partial(
        pl.kernel,
        out_type=jax.ShapeDtypeStruct((NC, NP, T1W), jnp.float32),
        mesh=_sc_mesh(),
        compiler_params=pltpu.CompilerParams(
            needs_layout_passes=False, use_tc_tiling_on_sc=False),
        scratch_types=[
            pltpu.VMEM_SHARED((NP, T1W), jnp.float32),
            pltpu.VMEM((nb, CH), jnp.int32),
            pltpu.VMEM((nb, CH), jnp.int32),
            pltpu.VMEM((2, CH, T1W), jnp.float32),
            pltpu.VMEM((2, CH, ADW), jnp.float32),
            pltpu.VMEM((2, CH, T1W), jnp.float32),
            pltpu.VMEM((16,), jnp.float32),
            pltpu.VMEM((CH, 16), jnp.float32),
            pltpu.VMEM((ZR, T1W), jnp.float32),
            pltpu.SemaphoreType.DMA((2,)),
            pltpu.SemaphoreType.DMA((2,)),
        ],
    )
    def sc1(src_hbm, dst_hbm, t1_hbm, ad_hbm, m_hbm, out_hbm,
            acc, idxs_all, idxd_all, rows, adst, contrib, mv, wtmp, zbuf,
            semg, semsc):
        cid = lax.axis_index("c")
        sid = lax.axis_index("s")
        _zero_acc(acc, zbuf, sid, T1W)

        pltpu.sync_copy(m_hbm, mv)
        io = lax.iota(jnp.int32, 16)
        hio = jnp.where(io < 8, io, io - 8)
        ca = plsc.load_gather(mv, [hio])
        cb = plsc.load_gather(mv, [hio + 8])
        z0 = ca + cb
        cvec = jnp.where(z0 > 0, z0, z0 * 0.2)

        plsc.subcore_barrier()

        def compute_chunk(b):
            @plsc.parallel_loop(0, CH, 1, unroll=4)
            def _(e):
                za = rows[b, e, pl.ds(HID, 16)]
                zb = adst[b, e, pl.ds(0, 16)]
                z = za + zb
                lr = jnp.where(z > 0, z, z * 0.2)
                w = jnp.exp(lr - cvec)
                contrib[b, e, pl.ds(HID, 16)] = w
                for v in range(4):
                    contrib[b, e, pl.ds(16 * v, 16)] = (
                        rows[b, e, pl.ds(16 * v, 16)] * w)

        rowbase = cid * (ep // NC // CH) + sid * nb
        _sc_pipeline(nb, rowbase, src_hbm, dst_hbm, t1_hbm, ad_hbm, acc,
                     idxs_all, idxd_all, rows, adst, contrib, semg, semsc,
                     compute_chunk)
        plsc.subcore_barrier()
        _writeback(acc, out_hbm, cid, sid)

    return sc1


def _make_sc2(ep):
    per_tile = ep // (NC * NS)
    nb = per_tile // CH

    @functools.partial(
        pl.kernel,
        out_type=jax.ShapeDtypeStruct((NC, NP, T2W), jnp.float32),
        mesh=_sc_mesh(),
        compiler_params=pltpu.CompilerParams(
            needs_layout_passes=False, use_tc_tiling_on_sc=False),
        scratch_types=[
            pltpu.VMEM_SHARED((NP, T2W), jnp.float32),
            pltpu.VMEM((nb, CH), jnp.int32),
            pltpu.VMEM((nb, CH), jnp.int32),
            pltpu.VMEM((2, CH, T2R), jnp.float32),
            pltpu.VMEM((2, CH, ADW), jnp.float32),
            pltpu.VMEM((2, CH, T2W), jnp.float32),
            pltpu.VMEM((16,), jnp.float32),
            pltpu.VMEM((CH, 16), jnp.float32),
            pltpu.VMEM((ZR, T2W), jnp.float32),
            pltpu.SemaphoreType.DMA((2,)),
            pltpu.SemaphoreType.DMA((2,)),
        ],
    )
    def sc2(src_hbm, dst_hbm, t2_hbm, ad_hbm, m_hbm, out_hbm,
            acc, idxs_all, idxd_all, rows, adst, contrib, mv, wtmp, zbuf,
            semg, semsc):
        cid = lax.axis_index("c")
        sid = lax.axis_index("s")
        _zero_acc(acc, zbuf, sid, T2W)

        pltpu.sync_copy(m_hbm, mv)
        io = lax.iota(jnp.int32, 16)
        ms = plsc.load_gather(mv, [jnp.zeros((16,), jnp.int32)])
        md = plsc.load_gather(mv, [jnp.full((16,), 8, jnp.int32)])
        z0 = ms + md
        cvec = jnp.where(z0 > 0, z0, z0 * 0.2)

        plsc.subcore_barrier()

        def compute_chunk(b):
            @plsc.parallel_loop(0, CH, 1, unroll=4)
            def _(e):
                za = rows[b, e, pl.ds(32, 16)]
                zb = adst[b, e, pl.ds(0, 16)]
                z = za + zb
                lr = jnp.where(z > 0, z, z * 0.2)
                w0 = jnp.exp(lr - cvec)
                wtmp[e, pl.ds(0, 16)] = w0
                w = plsc.load_gather(wtmp.at[e], [jnp.full((16,), 8, jnp.int32)])
                contrib[b, e, pl.ds(0, 16)] = rows[b, e, pl.ds(0, 16)] * w
                contrib[b, e, pl.ds(16, 16)] = rows[b, e, pl.ds(16, 16)] * w
                v2 = jnp.where(io < 8, rows[b, e, pl.ds(32, 16)] * w, w)
                contrib[b, e, pl.ds(32, 16)] = v2

        rowbase = cid * (ep // NC // CH) + sid * nb
        _sc_pipeline(nb, rowbase, src_hbm, dst_hbm, t2_hbm, ad_hbm, acc,
                     idxs_all, idxd_all, rows, adst, contrib, semg, semsc,
                     compute_chunk)
        plsc.subcore_barrier()
        _writeback(acc, out_hbm, cid, sid)

    return sc2


def kernel(x, edge_index, W1, att_src1, att_dst1, bias1, gamma1, beta1,
           W2, att_src2, att_dst2, bias2, gamma2, beta2):
    n, e = x.shape[0], edge_index.shape[1]
    loop = jnp.arange(n, dtype=edge_index.dtype)
    src = jnp.concatenate([edge_index[0], loop])
    dst = jnp.concatenate([edge_index[1], loop])
    el = e + n
    unit = NC * NS * CH
    ep = ((el + unit - 1) // unit) * unit
    pad = jnp.full((ep - el,), NP - 1, edge_index.dtype)
    srcp = jnp.concatenate([src, pad]).reshape(-1, CH)
    dstp = jnp.concatenate([dst, pad]).reshape(-1, CH)

    x_pad = jnp.pad(x, ((0, NP - n), (0, 0)))

    q = jnp.arange(HID)
    cq = (q % H1) * 8 + q // H1
    selp = ((q % H1)[:, None] ==
            jnp.arange(H1)[None, :]).astype(jnp.float32)
    w1p = W1[:, cq]
    am = att_src1.reshape(HID)[cq][:, None] * selp
    bm = att_dst1.reshape(HID)[cq][:, None] * selp
    selp8 = selp.T
    as2t = jnp.tile(att_src2.reshape(NCLS, 1), (1, 16))
    ad2t = jnp.tile(att_dst2.reshape(NCLS, 1), (1, 16))

    t1, ad1, m1 = _tc1(x_pad, w1p, am, bm)
    pacc1 = _make_sc1(ep)(srcp, dstp, t1, ad1, m1.reshape(16))
    t2, ad2, m2 = _tc2(pacc1, bias1[cq].reshape(1, HID),
                       gamma1[cq].reshape(1, HID),
                       beta1[cq].reshape(1, HID), W2[cq, :], as2t, ad2t,
                       selp8)
    _DBG = False
    if _DBG:
        m2f = m2.reshape(16)
        z02 = m2f[0] + m2f[8]
        c2 = jnp.where(z02 > 0, z02, 0.2 * z02)
        zdbg = t2[:, 48][src] + ad2[:, 0][dst]
        wdbg = jnp.exp(jnp.where(zdbg > 0, zdbg, 0.2 * zdbg) - c2)
        num2 = jax.ops.segment_sum(t2[:, :40][src] * wdbg[:, None], dst,
                                   num_segments=NP)
        den2 = jax.ops.segment_sum(wdbg, dst, num_segments=NP)
        acc2 = jnp.concatenate([num2, jnp.tile(den2[:, None], (1, 8))], axis=1)
        pacc2 = jnp.stack([acc2, jnp.zeros_like(acc2)])
    else:
        pacc2 = _make_sc2(ep)(srcp, dstp, t2, ad2, m2.reshape(16))
    out = _tc3(pacc2, bias2.reshape(1, NCLS), gamma2.reshape(1, NCLS),
               beta2.reshape(1, NCLS))
    return out[:N]
